# Initial kernel scaffold; baseline (speedup 1.0000x reference)
import jax
import jax.numpy as jnp
from jax.experimental import pallas as pl


def kernel(x, in_node, in_hedge, W, b, attn_node, attn_edge):
    raise NotImplementedError("write your pallas kernel here")


# speedup vs baseline: 21.2498x; 21.2498x over previous
import functools

import jax
import jax.numpy as jnp
from jax import lax
from jax.experimental import pallas as pl
from jax.experimental.pallas import tpu as pltpu
from jax.experimental.pallas import tpu_sc as plsc

N, M, E, D = 10000, 2000, 320000, 128
NC, NS, L = 2, 16, 16
NW = NC * NS
C = 128
SB = 8

TR1 = 10112
TR2 = 2048
ACC1 = 2048
ACC2 = 10112
OUT1 = ACC1
OUT2 = ACC2

K_CH = ((E + NW * C - 1) // (NW * C) + SB - 1) // SB * SB
EP = NW * C * K_CH
NSUP = K_CH // SB


def _sc_pass(tab_rows, sca_tab, scb_tab, gidx2d, sidx2d, *, acc_rows,
             out_rows, tr):
  ur = scb_tab.shape[0]
  opr = out_rows // NS
  spr = tr // NS
  nzb = acc_rows // C
  nzb_per = (nzb + NS - 1) // NS

  mesh = plsc.VectorSubcoreMesh(core_axis_name="c", subcore_axis_name="s",
                                num_cores=NC, num_subcores=NS)

  def body(tab_hbm, sca_hbm, scb_hbm, gidx_hbm, sidx_hbm,
           out_hbm, zout_hbm,
           idxg_v, idxs_v, rows_v, sca_v, scb_v, zred_v, acc_sh, tab_sh,
           sem):
    cid = lax.axis_index("c")
    sid = lax.axis_index("s")
    wid = cid * NS + sid

    pltpu.sync_copy(tab_hbm.at[pl.ds(sid * spr, spr)],
                    tab_sh.at[pl.ds(sid * spr, spr)])
    pltpu.sync_copy(sca_hbm, sca_v)
    pltpu.sync_copy(scb_hbm, scb_v)

    zv = jnp.zeros((L,), jnp.float32)

    def zrow(r, _):
      for j in range(D // L):
        rows_v[r, pl.ds(j * L, L)] = zv
      return 0

    lax.fori_loop(0, C, zrow, 0)

    def zblk(i, _):
      b = i * NS + sid

      @pl.when(b < nzb)
      def _():
        pltpu.sync_copy(rows_v, acc_sh.at[pl.ds(b * C, C)])
      return 0

    lax.fori_loop(0, nzb_per, zblk, 0)

    plsc.subcore_barrier()

    crow0 = wid * K_CH

    def sup(u, zacc):
      cr = crow0 + u * SB
      pltpu.sync_copy(gidx_hbm.at[pl.ds(cr, SB)], idxg_v)
      pltpu.sync_copy(sidx_hbm.at[pl.ds(cr, SB)], idxs_v)
      for j in range(SB):
        pltpu.async_copy(tab_sh.at[idxg_v.at[j]], rows_v, sem).wait()
        pltpu.sync_copy(rows_v, acc_sh.at[idxs_v.at[j]], add=True)
        for h in range(C // L):
          ig = idxg_v[j, pl.ds(h * L, L)]
          isx = idxs_v[j, pl.ds(h * L, L)]
          va = plsc.load_gather(sca_v, [ig])
          vb = plsc.load_gather(scb_v, [isx])
          zacc = zacc + va * vb
      return zacc

    z = lax.fori_loop(0, NSUP, sup, jnp.zeros((L,), jnp.float32))

    plsc.subcore_barrier()

    zred_v[...] = z
    pltpu.sync_copy(zred_v, zout_hbm.at[wid])
    pltpu.sync_copy(acc_sh.at[pl.ds(sid * opr, opr)],
                    out_hbm.at[cid, pl.ds(sid * opr, opr)])

  call = pl.kernel(
      body,
      out_type=(jax.ShapeDtypeStruct((NC, out_rows, D), jnp.float32),
                jax.ShapeDtypeStruct((NW, L), jnp.float32)),
      mesh=mesh,
      compiler_params=pltpu.CompilerParams(needs_layout_passes=False),
      scratch_types=(
          pltpu.VMEM((SB, C), jnp.int32),
          pltpu.VMEM((SB, C), jnp.int32),
          pltpu.VMEM((C, D), jnp.float32),
          pltpu.VMEM((tr,), jnp.float32),
          pltpu.VMEM((ur,), jnp.float32),
          pltpu.VMEM((L,), jnp.float32),
          pltpu.VMEM_SHARED((acc_rows, D), jnp.float32),
          pltpu.VMEM_SHARED((tr, D), jnp.float32),
          pltpu.SemaphoreType.DMA,
      ),
  )
  return call(tab_rows, sca_tab, scb_tab, gidx2d, sidx2d)


def _k12_body(x_ref, w_ref, b_ref, an_ref, y_ref, p1_ref):
  xt = jnp.dot(x_ref[...], w_ref[...],
               preferred_element_type=jnp.float32) + b_ref[...]
  a = jnp.dot(xt, an_ref[...], preferred_element_type=jnp.float32)
  p1 = jnp.exp(a - jnp.max(a))
  y_ref[: N, :] = p1 * xt
  y_ref[N:, :] = jnp.zeros((TR1 - N, D), jnp.float32)
  p1_ref[: N, :] = p1
  p1_ref[N:, :] = jnp.zeros((TR1 - N, 1), jnp.float32)


def _k3_body(hp_ref, z1_ref, ae_ref, z_ref, p2_ref):
  h = hp_ref[0, : M, :] + hp_ref[1, : M, :]
  z1 = jnp.sum(z1_ref[...])
  gu = jnp.dot(h, ae_ref[...], preferred_element_type=jnp.float32)
  p2 = jnp.exp((gu - jnp.max(gu)) / z1)
  z_ref[: M, :] = (p2 / z1) * h
  z_ref[M:, :] = jnp.zeros((TR2 - M, D), jnp.float32)
  p2_ref[: M, :] = p2
  p2_ref[M:, :] = jnp.zeros((TR2 - M, 1), jnp.float32)


def _k4_body(np_ref, z2_ref, p1_ref, o_ref):
  s = np_ref[0, : N, :] + np_ref[1, : N, :]
  z2 = jnp.sum(z2_ref[...])
  o_ref[...] = p1_ref[: N, :] * s * (1.0 / z2)


def kernel(x, in_node, in_hedge, W, b, attn_node, attn_edge):
  in_node = in_node.astype(jnp.int32)
  in_hedge = in_hedge.astype(jnp.int32)
  pad = EP - E
  ar = jnp.arange(pad, dtype=jnp.int32)
  g1 = jnp.concatenate([in_node, N + (ar % 16)]).reshape(EP // C, C)
  s1 = jnp.concatenate([in_hedge, jnp.zeros((pad,), jnp.int32)]
                       ).reshape(EP // C, C)
  g2 = jnp.concatenate([in_hedge, M + (ar % 16)]).reshape(EP // C, C)
  s2 = jnp.concatenate([in_node, jnp.zeros((pad,), jnp.int32)]
                       ).reshape(EP // C, C)

  y_pad, p1_pad = pl.pallas_call(
      _k12_body,
      out_shape=(jax.ShapeDtypeStruct((TR1, D), jnp.float32),
                 jax.ShapeDtypeStruct((TR1, 1), jnp.float32)),
  )(x, W, b.reshape(1, D), attn_node)

  ones_m = jnp.ones((TR2,), jnp.float32)
  h_part, z1_part = _sc_pass(y_pad, p1_pad.reshape(TR1), ones_m, g1, s1,
                             acc_rows=ACC1, out_rows=OUT1, tr=TR1)

  z_pad, p2_pad = pl.pallas_call(
      _k3_body,
      out_shape=(jax.ShapeDtypeStruct((TR2, D), jnp.float32),
                 jax.ShapeDtypeStruct((TR2, 1), jnp.float32)),
  )(h_part, z1_part, attn_edge)

  n_part, z2_part = _sc_pass(z_pad, p2_pad.reshape(TR2),
                             p1_pad.reshape(TR1), g2, s2,
                             acc_rows=ACC2, out_rows=OUT2, tr=TR2)

  h_n = pl.pallas_call(
      _k4_body,
      out_shape=jax.ShapeDtypeStruct((N, D), jnp.float32),
  )(n_part, z2_part, p1_pad)
  return h_n


# speedup vs baseline: 31.5326x; 1.4839x over previous
import functools

import jax
import jax.numpy as jnp
from jax import lax
from jax.experimental import pallas as pl
from jax.experimental.pallas import tpu as pltpu
from jax.experimental.pallas import tpu_sc as plsc

N, M, E, D = 10000, 2000, 320000, 128
NC, NS, L = 2, 16, 16
NW = NC * NS
C = 128
SB = 8

TR1 = 10112
TR2 = 2048
ACC1 = 2048
ACC2 = 10112
OUT1 = ACC1
OUT2 = ACC2

K_CH = ((E + NW * C - 1) // (NW * C) + SB - 1) // SB * SB
EP = NW * C * K_CH
NSUP = K_CH // SB


def _sc_pass(tab_rows, sca_tab, scb_tab, gidx2d, sidx2d, *, acc_rows,
             out_rows, tr, nbuf, stage_tab):
  ur = scb_tab.shape[0]
  opr = out_rows // NS
  spr = tr // NS
  nzb = acc_rows // C
  nzb_per = (nzb + NS - 1) // NS
  tab_sh_rows = tr if stage_tab else 16

  mesh = plsc.VectorSubcoreMesh(core_axis_name="c", subcore_axis_name="s",
                                num_cores=NC, num_subcores=NS)

  def body(tab_hbm, sca_hbm, scb_hbm, gidx_hbm, sidx_hbm,
           out_hbm, zout_hbm,
           idxg_v, idxs_v, rows_v, sca_v, scb_v, zred_v, acc_sh, tab_sh,
           g0, g1, g2, g3, s0, s1, s2, s3):
    gsems = (g0, g1, g2, g3)[:nbuf]
    ssems = (s0, s1, s2, s3)[:nbuf]
    gtab = tab_sh if stage_tab else tab_hbm
    cid = lax.axis_index("c")
    sid = lax.axis_index("s")
    wid = cid * NS + sid

    if stage_tab:
      pltpu.sync_copy(tab_hbm.at[pl.ds(sid * spr, spr)],
                      tab_sh.at[pl.ds(sid * spr, spr)])
    pltpu.sync_copy(sca_hbm, sca_v)
    pltpu.sync_copy(scb_hbm, scb_v)

    zv = jnp.zeros((L,), jnp.float32)

    def zrow(r, _):
      for j in range(D // L):
        rows_v[0, r, pl.ds(j * L, L)] = zv
      return 0

    lax.fori_loop(0, C, zrow, 0)

    def zblk(i, _):
      b = i * NS + sid

      @pl.when(b < nzb)
      def _():
        pltpu.sync_copy(rows_v.at[0], acc_sh.at[pl.ds(b * C, C)])
      return 0

    lax.fori_loop(0, nzb_per, zblk, 0)

    plsc.subcore_barrier()

    crow0 = wid * K_CH

    def sup(u, zacc):
      cr = crow0 + u * SB
      pltpu.sync_copy(gidx_hbm.at[pl.ds(cr, SB)], idxg_v)
      pltpu.sync_copy(sidx_hbm.at[pl.ds(cr, SB)], idxs_v)
      dg = [None] * SB
      ds = [None] * SB
      for j in range(SB + 1):
        if j < SB:
          b = j % nbuf
          if j >= nbuf:
            ds[j - nbuf].wait()
          dg[j] = pltpu.async_copy(gtab.at[idxg_v.at[j]],
                                   rows_v.at[b], gsems[b])
        if j >= 1:
          jj = j - 1
          bb = jj % nbuf
          dg[jj].wait()
          ds[jj] = pltpu.async_copy(rows_v.at[bb],
                                    acc_sh.at[idxs_v.at[jj]],
                                    ssems[bb], add=True)
          for h in range(C // L):
            ig = idxg_v[jj, pl.ds(h * L, L)]
            isx = idxs_v[jj, pl.ds(h * L, L)]
            va = plsc.load_gather(sca_v, [ig])
            vb = plsc.load_gather(scb_v, [isx])
            zacc = zacc + va * vb
      for j in range(SB - nbuf, SB):
        ds[j].wait()
      return zacc

    z = lax.fori_loop(0, NSUP, sup, jnp.zeros((L,), jnp.float32))

    plsc.subcore_barrier()

    zred_v[...] = z
    pltpu.sync_copy(zred_v, zout_hbm.at[wid])
    pltpu.sync_copy(acc_sh.at[pl.ds(sid * opr, opr)],
                    out_hbm.at[cid, pl.ds(sid * opr, opr)])

  call = pl.kernel(
      body,
      out_type=(jax.ShapeDtypeStruct((NC, out_rows, D), jnp.float32),
                jax.ShapeDtypeStruct((NW, L), jnp.float32)),
      mesh=mesh,
      compiler_params=pltpu.CompilerParams(needs_layout_passes=False),
      scratch_types=(
          pltpu.VMEM((SB, C), jnp.int32),
          pltpu.VMEM((SB, C), jnp.int32),
          pltpu.VMEM((nbuf, C, D), jnp.float32),
          pltpu.VMEM((tr,), jnp.float32),
          pltpu.VMEM((ur,), jnp.float32),
          pltpu.VMEM((L,), jnp.float32),
          pltpu.VMEM_SHARED((acc_rows, D), jnp.float32),
          pltpu.VMEM_SHARED((tab_sh_rows, D), jnp.float32),
          pltpu.SemaphoreType.DMA,
          pltpu.SemaphoreType.DMA,
          pltpu.SemaphoreType.DMA,
          pltpu.SemaphoreType.DMA,
          pltpu.SemaphoreType.DMA,
          pltpu.SemaphoreType.DMA,
          pltpu.SemaphoreType.DMA,
          pltpu.SemaphoreType.DMA,
      ),
  )
  return call(tab_rows, sca_tab, scb_tab, gidx2d, sidx2d)


def _k12_body(x_ref, w_ref, b_ref, an_ref, y_ref, p1_ref):
  xt = jnp.dot(x_ref[...], w_ref[...],
               preferred_element_type=jnp.float32) + b_ref[...]
  a = jnp.dot(xt, an_ref[...], preferred_element_type=jnp.float32)
  p1 = jnp.exp(a - jnp.max(a))
  y_ref[: N, :] = p1 * xt
  y_ref[N:, :] = jnp.zeros((TR1 - N, D), jnp.float32)
  p1_ref[: N, :] = p1
  p1_ref[N:, :] = jnp.zeros((TR1 - N, 1), jnp.float32)


def _k3_body(hp_ref, z1_ref, ae_ref, z_ref, p2_ref):
  h = hp_ref[0, : M, :] + hp_ref[1, : M, :]
  z1 = jnp.sum(z1_ref[...])
  gu = jnp.dot(h, ae_ref[...], preferred_element_type=jnp.float32)
  p2 = jnp.exp((gu - jnp.max(gu)) / z1)
  z_ref[: M, :] = (p2 / z1) * h
  z_ref[M:, :] = jnp.zeros((TR2 - M, D), jnp.float32)
  p2_ref[: M, :] = p2
  p2_ref[M:, :] = jnp.zeros((TR2 - M, 1), jnp.float32)


def _k4_body(np_ref, z2_ref, p1_ref, o_ref):
  s = np_ref[0, : N, :] + np_ref[1, : N, :]
  z2 = jnp.sum(z2_ref[...])
  o_ref[...] = p1_ref[: N, :] * s * (1.0 / z2)


def kernel(x, in_node, in_hedge, W, b, attn_node, attn_edge):
  in_node = in_node.astype(jnp.int32)
  in_hedge = in_hedge.astype(jnp.int32)
  pad = EP - E
  ar = jnp.arange(pad, dtype=jnp.int32)
  g1 = jnp.concatenate([in_node, N + (ar % 16)]).reshape(EP // C, C)
  s1 = jnp.concatenate([in_hedge, jnp.zeros((pad,), jnp.int32)]
                       ).reshape(EP // C, C)
  g2 = jnp.concatenate([in_hedge, M + (ar % 16)]).reshape(EP // C, C)
  s2 = jnp.concatenate([in_node, jnp.zeros((pad,), jnp.int32)]
                       ).reshape(EP // C, C)

  y_pad, p1_pad = pl.pallas_call(
      _k12_body,
      out_shape=(jax.ShapeDtypeStruct((TR1, D), jnp.float32),
                 jax.ShapeDtypeStruct((TR1, 1), jnp.float32)),
  )(x, W, b.reshape(1, D), attn_node)

  ones_m = jnp.ones((TR2,), jnp.float32)
  h_part, z1_part = _sc_pass(y_pad, p1_pad.reshape(TR1), ones_m, g1, s1,
                             acc_rows=ACC1, out_rows=OUT1, tr=TR1,
                             nbuf=4, stage_tab=False)

  z_pad, p2_pad = pl.pallas_call(
      _k3_body,
      out_shape=(jax.ShapeDtypeStruct((TR2, D), jnp.float32),
                 jax.ShapeDtypeStruct((TR2, 1), jnp.float32)),
  )(h_part, z1_part, attn_edge)

  n_part, z2_part = _sc_pass(z_pad, p2_pad.reshape(TR2),
                             p1_pad.reshape(TR1), g2, s2,
                             acc_rows=ACC2, out_rows=OUT2, tr=TR2,
                             nbuf=2, stage_tab=False)

  h_n = pl.pallas_call(
      _k4_body,
      out_shape=jax.ShapeDtypeStruct((N, D), jnp.float32),
  )(n_part, z2_part, p1_pad)
  return h_n


# speedup vs baseline: 33.3610x; 1.0580x over previous
import functools

import jax
import jax.numpy as jnp
from jax import lax
from jax.experimental import pallas as pl
from jax.experimental.pallas import tpu as pltpu
from jax.experimental.pallas import tpu_sc as plsc

N, M, E, D = 10000, 2000, 320000, 128
NC, NS, L = 2, 16, 16
NW = NC * NS
C = 128
SB = 8

TR1 = 10112
TR2 = 2048
ACC1 = 2048
ACC2 = 10112
OUT1 = ACC1
OUT2 = ACC2

K_CH = ((E + NW * C - 1) // (NW * C) + SB - 1) // SB * SB
EP = NW * C * K_CH
NSUP = K_CH // SB


def _sc_pass(tab_rows, sca_tab, scb_tab, gidx2d, sidx2d, *, acc_rows,
             out_rows, tr, nbuf, stage_tab):
  ur = scb_tab.shape[0]
  opr = out_rows // NS
  spr = tr // NS
  nzb = acc_rows // C
  nzb_per = (nzb + NS - 1) // NS
  tab_sh_rows = tr if stage_tab else 16

  mesh = plsc.VectorSubcoreMesh(core_axis_name="c", subcore_axis_name="s",
                                num_cores=NC, num_subcores=NS)

  def body(tab_hbm, sca_hbm, scb_hbm, gidx_hbm, sidx_hbm,
           out_hbm, zout_hbm,
           idxg_v, idxs_v, rows_v, sca_v, scb_v, zred_v, acc_sh, tab_sh,
           isem, g0, g1, g2, g3, s0, s1, s2, s3):
    gsems = (g0, g1, g2, g3)[:nbuf]
    ssems = (s0, s1, s2, s3)[:nbuf]
    gtab = tab_sh if stage_tab else tab_hbm
    cid = lax.axis_index("c")
    sid = lax.axis_index("s")
    wid = cid * NS + sid

    if stage_tab:
      pltpu.sync_copy(tab_hbm.at[pl.ds(sid * spr, spr)],
                      tab_sh.at[pl.ds(sid * spr, spr)])
    pltpu.sync_copy(sca_hbm, sca_v)
    pltpu.sync_copy(scb_hbm, scb_v)

    zv = jnp.zeros((L,), jnp.float32)

    def zrow(r, _):
      for j in range(D // L):
        rows_v[0, r, pl.ds(j * L, L)] = zv
      return 0

    lax.fori_loop(0, C, zrow, 0)

    def zblk(i, _):
      b = i * NS + sid

      @pl.when(b < nzb)
      def _():
        pltpu.sync_copy(rows_v.at[0], acc_sh.at[pl.ds(b * C, C)])
      return 0

    lax.fori_loop(0, nzb_per, zblk, 0)

    plsc.subcore_barrier()

    crow0 = wid * K_CH

    pltpu.async_copy(gidx_hbm.at[pl.ds(crow0, SB)], idxg_v.at[0], isem)
    pltpu.async_copy(sidx_hbm.at[pl.ds(crow0, SB)], idxs_v.at[0], isem)

    def sup(u, zacc):
      pu = lax.rem(u, 2)
      cr = crow0 + u * SB
      pltpu.make_async_copy(gidx_hbm.at[pl.ds(cr, SB)],
                            idxg_v.at[pu], isem).wait()
      pltpu.make_async_copy(sidx_hbm.at[pl.ds(cr, SB)],
                            idxs_v.at[pu], isem).wait()
      crn = crow0 + lax.rem(u + 1, NSUP) * SB
      pltpu.async_copy(gidx_hbm.at[pl.ds(crn, SB)], idxg_v.at[1 - pu], isem)
      pltpu.async_copy(sidx_hbm.at[pl.ds(crn, SB)], idxs_v.at[1 - pu], isem)
      dg = [None] * SB
      ds = [None] * SB
      for j in range(SB + 1):
        if j < SB:
          b = j % nbuf
          if j >= nbuf:
            ds[j - nbuf].wait()
          dg[j] = pltpu.async_copy(gtab.at[idxg_v.at[pu, j]],
                                   rows_v.at[b], gsems[b])
        if j >= 1:
          jj = j - 1
          bb = jj % nbuf
          dg[jj].wait()
          ds[jj] = pltpu.async_copy(rows_v.at[bb],
                                    acc_sh.at[idxs_v.at[pu, jj]],
                                    ssems[bb], add=True)
          for h in range(C // L):
            ig = idxg_v[pu, jj, pl.ds(h * L, L)]
            isx = idxs_v[pu, jj, pl.ds(h * L, L)]
            va = plsc.load_gather(sca_v, [ig])
            vb = plsc.load_gather(scb_v, [isx])
            zacc = zacc + va * vb
      for j in range(SB - nbuf, SB):
        ds[j].wait()
      return zacc

    z = lax.fori_loop(0, NSUP, sup, jnp.zeros((L,), jnp.float32))
    pltpu.make_async_copy(gidx_hbm.at[pl.ds(crow0, SB)],
                          idxg_v.at[NSUP % 2], isem).wait()
    pltpu.make_async_copy(sidx_hbm.at[pl.ds(crow0, SB)],
                          idxs_v.at[NSUP % 2], isem).wait()

    plsc.subcore_barrier()

    zred_v[...] = z
    pltpu.sync_copy(zred_v, zout_hbm.at[wid])
    pltpu.sync_copy(acc_sh.at[pl.ds(sid * opr, opr)],
                    out_hbm.at[cid, pl.ds(sid * opr, opr)])

  call = pl.kernel(
      body,
      out_type=(jax.ShapeDtypeStruct((NC, out_rows, D), jnp.float32),
                jax.ShapeDtypeStruct((NW, L), jnp.float32)),
      mesh=mesh,
      compiler_params=pltpu.CompilerParams(needs_layout_passes=False),
      scratch_types=(
          pltpu.VMEM((2, SB, C), jnp.int32),
          pltpu.VMEM((2, SB, C), jnp.int32),
          pltpu.VMEM((nbuf, C, D), jnp.float32),
          pltpu.VMEM((tr,), jnp.float32),
          pltpu.VMEM((ur,), jnp.float32),
          pltpu.VMEM((L,), jnp.float32),
          pltpu.VMEM_SHARED((acc_rows, D), jnp.float32),
          pltpu.VMEM_SHARED((tab_sh_rows, D), jnp.float32),
          pltpu.SemaphoreType.DMA,
          pltpu.SemaphoreType.DMA,
          pltpu.SemaphoreType.DMA,
          pltpu.SemaphoreType.DMA,
          pltpu.SemaphoreType.DMA,
          pltpu.SemaphoreType.DMA,
          pltpu.SemaphoreType.DMA,
          pltpu.SemaphoreType.DMA,
          pltpu.SemaphoreType.DMA,
      ),
  )
  return call(tab_rows, sca_tab, scb_tab, gidx2d, sidx2d)


def _k12_body(x_ref, w_ref, b_ref, an_ref, y_ref, p1_ref):
  xt = jnp.dot(x_ref[...], w_ref[...],
               preferred_element_type=jnp.float32) + b_ref[...]
  a = jnp.dot(xt, an_ref[...], preferred_element_type=jnp.float32)
  p1 = jnp.exp(a - jnp.max(a))
  y_ref[: N, :] = p1 * xt
  y_ref[N:, :] = jnp.zeros((TR1 - N, D), jnp.float32)
  p1_ref[: N, :] = p1
  p1_ref[N:, :] = jnp.zeros((TR1 - N, 1), jnp.float32)


def _k3_body(hp_ref, z1_ref, ae_ref, z_ref, p2_ref):
  h = hp_ref[0, : M, :] + hp_ref[1, : M, :]
  z1 = jnp.sum(z1_ref[...])
  gu = jnp.dot(h, ae_ref[...], preferred_element_type=jnp.float32)
  p2 = jnp.exp((gu - jnp.max(gu)) / z1)
  z_ref[: M, :] = (p2 / z1) * h
  z_ref[M:, :] = jnp.zeros((TR2 - M, D), jnp.float32)
  p2_ref[: M, :] = p2
  p2_ref[M:, :] = jnp.zeros((TR2 - M, 1), jnp.float32)


def _k4_body(np_ref, z2_ref, p1_ref, o_ref):
  s = np_ref[0, : N, :] + np_ref[1, : N, :]
  z2 = jnp.sum(z2_ref[...])
  o_ref[...] = p1_ref[: N, :] * s * (1.0 / z2)


def kernel(x, in_node, in_hedge, W, b, attn_node, attn_edge):
  in_node = in_node.astype(jnp.int32)
  in_hedge = in_hedge.astype(jnp.int32)
  pad = EP - E
  ar = jnp.arange(pad, dtype=jnp.int32)
  g1 = jnp.concatenate([in_node, N + (ar % 16)]).reshape(EP // C, C)
  s1 = jnp.concatenate([in_hedge, ar % ACC1]).reshape(EP // C, C)
  g2 = jnp.concatenate([in_hedge, M + (ar % 16)]).reshape(EP // C, C)
  s2 = jnp.concatenate([in_node, ar % ACC2]).reshape(EP // C, C)

  y_pad, p1_pad = pl.pallas_call(
      _k12_body,
      out_shape=(jax.ShapeDtypeStruct((TR1, D), jnp.float32),
                 jax.ShapeDtypeStruct((TR1, 1), jnp.float32)),
  )(x, W, b.reshape(1, D), attn_node)

  ones_m = jnp.ones((TR2,), jnp.float32)
  h_part, z1_part = _sc_pass(y_pad, p1_pad.reshape(TR1), ones_m, g1, s1,
                             acc_rows=ACC1, out_rows=OUT1, tr=TR1,
                             nbuf=4, stage_tab=False)

  z_pad, p2_pad = pl.pallas_call(
      _k3_body,
      out_shape=(jax.ShapeDtypeStruct((TR2, D), jnp.float32),
                 jax.ShapeDtypeStruct((TR2, 1), jnp.float32)),
  )(h_part, z1_part, attn_edge)

  n_part, z2_part = _sc_pass(z_pad, p2_pad.reshape(TR2),
                             p1_pad.reshape(TR1), g2, s2,
                             acc_rows=ACC2, out_rows=OUT2, tr=TR2,
                             nbuf=2, stage_tab=False)

  h_n = pl.pallas_call(
      _k4_body,
      out_shape=jax.ShapeDtypeStruct((N, D), jnp.float32),
  )(n_part, z2_part, p1_pad)
  return h_n


# speedup vs baseline: 35.0204x; 1.0497x over previous
import functools

import jax
import jax.numpy as jnp
from jax import lax
from jax.experimental import pallas as pl
from jax.experimental.pallas import tpu as pltpu
from jax.experimental.pallas import tpu_sc as plsc

N, M, E, D = 10000, 2000, 320000, 128
NC, NS, L = 2, 16, 16
NW = NC * NS
C = 128
SB = 8

TR1 = 10112
TR2 = 2048
ACC1 = 2048
ACC2 = 10112
OUT1 = ACC1
OUT2 = ACC2

K_CH = ((E + NW * C - 1) // (NW * C) + SB - 1) // SB * SB
EP = NW * C * K_CH
NSUP = K_CH // SB


def _sc_pass(tab_rows, sca_tab, scb_tab, gidx2d, sidx2d, *, acc_rows,
             out_rows, tr, nbuf, stage_tab, scb_is_ones=False):
  ur = scb_tab.shape[0]
  opr = out_rows // NS
  spr = tr // NS
  nzb = acc_rows // C
  nzb_per = (nzb + NS - 1) // NS
  tab_sh_rows = tr if stage_tab else 16

  mesh = plsc.VectorSubcoreMesh(core_axis_name="c", subcore_axis_name="s",
                                num_cores=NC, num_subcores=NS)

  def body(tab_hbm, sca_hbm, scb_hbm, gidx_hbm, sidx_hbm,
           out_hbm, zout_hbm,
           idxg_v, idxs_v, rows_v, sca_v, scb_v, zred_v, acc_sh, tab_sh,
           isem, g0, g1, g2, g3, s0, s1, s2, s3):
    gsems = (g0, g1, g2, g3)[:nbuf]
    ssems = (s0, s1, s2, s3)[:nbuf]
    gtab = tab_sh if stage_tab else tab_hbm
    cid = lax.axis_index("c")
    sid = lax.axis_index("s")
    wid = cid * NS + sid

    if stage_tab:
      pltpu.sync_copy(tab_hbm.at[pl.ds(sid * spr, spr)],
                      tab_sh.at[pl.ds(sid * spr, spr)])
    pltpu.sync_copy(sca_hbm, sca_v)
    pltpu.sync_copy(scb_hbm, scb_v)

    zv = jnp.zeros((L,), jnp.float32)

    def zrow(r, _):
      for j in range(D // L):
        rows_v[0, r, pl.ds(j * L, L)] = zv
      return 0

    lax.fori_loop(0, C, zrow, 0)

    def zblk(i, _):
      b = i * NS + sid

      @pl.when(b < nzb)
      def _():
        pltpu.sync_copy(rows_v.at[0], acc_sh.at[pl.ds(b * C, C)])
      return 0

    lax.fori_loop(0, nzb_per, zblk, 0)

    plsc.subcore_barrier()

    crow0 = wid * K_CH

    pltpu.async_copy(gidx_hbm.at[pl.ds(crow0, SB)], idxg_v.at[0], isem)
    pltpu.async_copy(sidx_hbm.at[pl.ds(crow0, SB)], idxs_v.at[0], isem)

    def sup(u, zacc):
      pu = lax.rem(u, 2)
      cr = crow0 + u * SB
      pltpu.make_async_copy(gidx_hbm.at[pl.ds(cr, SB)],
                            idxg_v.at[pu], isem).wait()
      pltpu.make_async_copy(sidx_hbm.at[pl.ds(cr, SB)],
                            idxs_v.at[pu], isem).wait()
      dg = [None] * SB
      for j in range(SB + 1):
        if j < SB:
          b = j % nbuf
          if j >= nbuf:
            pltpu.make_async_copy(rows_v.at[b],
                                  acc_sh.at[idxs_v.at[pu, j - nbuf]],
                                  ssems[b]).wait()
          else:
            @pl.when(u > 0)
            def _():
              pltpu.make_async_copy(rows_v.at[b],
                                    acc_sh.at[idxs_v.at[pu, j]],
                                    ssems[b]).wait()
          dg[j] = pltpu.async_copy(gtab.at[idxg_v.at[pu, j]],
                                   rows_v.at[b], gsems[b])
        if j == nbuf:
          crn = crow0 + lax.rem(u + 1, NSUP) * SB
          pltpu.async_copy(gidx_hbm.at[pl.ds(crn, SB)],
                           idxg_v.at[1 - pu], isem)
          pltpu.async_copy(sidx_hbm.at[pl.ds(crn, SB)],
                           idxs_v.at[1 - pu], isem)
        if j >= 1:
          jj = j - 1
          bb = jj % nbuf
          dg[jj].wait()
          pltpu.async_copy(rows_v.at[bb], acc_sh.at[idxs_v.at[pu, jj]],
                           ssems[bb], add=True)
          for h in range(C // L):
            ig = idxg_v[pu, jj, pl.ds(h * L, L)]
            va = plsc.load_gather(sca_v, [ig])
            if scb_is_ones:
              zacc = zacc + va
            else:
              isx = idxs_v[pu, jj, pl.ds(h * L, L)]
              vb = plsc.load_gather(scb_v, [isx])
              zacc = zacc + va * vb
      return zacc

    z = lax.fori_loop(0, NSUP, sup, jnp.zeros((L,), jnp.float32))
    lp = (NSUP - 1) % 2
    for j in range(SB - nbuf, SB):
      pltpu.make_async_copy(rows_v.at[j % nbuf],
                            acc_sh.at[idxs_v.at[lp, j]],
                            ssems[j % nbuf]).wait()
    pltpu.make_async_copy(gidx_hbm.at[pl.ds(crow0, SB)],
                          idxg_v.at[NSUP % 2], isem).wait()
    pltpu.make_async_copy(sidx_hbm.at[pl.ds(crow0, SB)],
                          idxs_v.at[NSUP % 2], isem).wait()

    plsc.subcore_barrier()

    zred_v[...] = z
    pltpu.sync_copy(zred_v, zout_hbm.at[wid])
    pltpu.sync_copy(acc_sh.at[pl.ds(sid * opr, opr)],
                    out_hbm.at[cid, pl.ds(sid * opr, opr)])

  call = pl.kernel(
      body,
      out_type=(jax.ShapeDtypeStruct((NC, out_rows, D), jnp.float32),
                jax.ShapeDtypeStruct((NW, L), jnp.float32)),
      mesh=mesh,
      compiler_params=pltpu.CompilerParams(needs_layout_passes=False),
      scratch_types=(
          pltpu.VMEM((2, SB, C), jnp.int32),
          pltpu.VMEM((2, SB, C), jnp.int32),
          pltpu.VMEM((nbuf, C, D), jnp.float32),
          pltpu.VMEM((tr,), jnp.float32),
          pltpu.VMEM((ur,), jnp.float32),
          pltpu.VMEM((L,), jnp.float32),
          pltpu.VMEM_SHARED((acc_rows, D), jnp.float32),
          pltpu.VMEM_SHARED((tab_sh_rows, D), jnp.float32),
          pltpu.SemaphoreType.DMA,
          pltpu.SemaphoreType.DMA,
          pltpu.SemaphoreType.DMA,
          pltpu.SemaphoreType.DMA,
          pltpu.SemaphoreType.DMA,
          pltpu.SemaphoreType.DMA,
          pltpu.SemaphoreType.DMA,
          pltpu.SemaphoreType.DMA,
          pltpu.SemaphoreType.DMA,
      ),
  )
  return call(tab_rows, sca_tab, scb_tab, gidx2d, sidx2d)


def _k12_body(x_ref, w_ref, b_ref, an_ref, y_ref, p1_ref):
  xt = jnp.dot(x_ref[...], w_ref[...],
               preferred_element_type=jnp.float32) + b_ref[...]
  a = jnp.dot(xt, an_ref[...], preferred_element_type=jnp.float32)
  p1 = jnp.exp(a - jnp.max(a))
  y_ref[: N, :] = p1 * xt
  y_ref[N:, :] = jnp.zeros((TR1 - N, D), jnp.float32)
  p1_ref[: N, :] = p1
  p1_ref[N:, :] = jnp.zeros((TR1 - N, 1), jnp.float32)


def _k3_body(hp_ref, z1_ref, ae_ref, z_ref, p2_ref):
  h = hp_ref[0, : M, :] + hp_ref[1, : M, :]
  z1 = jnp.sum(z1_ref[...])
  gu = jnp.dot(h, ae_ref[...], preferred_element_type=jnp.float32)
  p2 = jnp.exp((gu - jnp.max(gu)) / z1)
  z_ref[: M, :] = (p2 / z1) * h
  z_ref[M:, :] = jnp.zeros((TR2 - M, D), jnp.float32)
  p2_ref[: M, :] = p2
  p2_ref[M:, :] = jnp.zeros((TR2 - M, 1), jnp.float32)


def _k4_body(np_ref, z2_ref, p1_ref, o_ref):
  s = np_ref[0, : N, :] + np_ref[1, : N, :]
  z2 = jnp.sum(z2_ref[...])
  o_ref[...] = p1_ref[: N, :] * s * (1.0 / z2)


def kernel(x, in_node, in_hedge, W, b, attn_node, attn_edge):
  in_node = in_node.astype(jnp.int32)
  in_hedge = in_hedge.astype(jnp.int32)
  pad = EP - E
  ar = jnp.arange(pad, dtype=jnp.int32)
  g1 = jnp.concatenate([in_node, N + (ar % 16)]).reshape(EP // C, C)
  s1 = jnp.concatenate([in_hedge, ar % ACC1]).reshape(EP // C, C)
  g2 = jnp.concatenate([in_hedge, M + (ar % 16)]).reshape(EP // C, C)
  s2 = jnp.concatenate([in_node, ar % ACC2]).reshape(EP // C, C)

  y_pad, p1_pad = pl.pallas_call(
      _k12_body,
      out_shape=(jax.ShapeDtypeStruct((TR1, D), jnp.float32),
                 jax.ShapeDtypeStruct((TR1, 1), jnp.float32)),
  )(x, W, b.reshape(1, D), attn_node)

  ones_m = jnp.ones((TR2,), jnp.float32)
  h_part, z1_part = _sc_pass(y_pad, p1_pad.reshape(TR1), ones_m, g1, s1,
                             acc_rows=ACC1, out_rows=OUT1, tr=TR1,
                             nbuf=4, stage_tab=False, scb_is_ones=True)

  z_pad, p2_pad = pl.pallas_call(
      _k3_body,
      out_shape=(jax.ShapeDtypeStruct((TR2, D), jnp.float32),
                 jax.ShapeDtypeStruct((TR2, 1), jnp.float32)),
  )(h_part, z1_part, attn_edge)

  n_part, z2_part = _sc_pass(z_pad, p2_pad.reshape(TR2),
                             p1_pad.reshape(TR1), g2, s2,
                             acc_rows=ACC2, out_rows=OUT2, tr=TR2,
                             nbuf=2, stage_tab=False)

  h_n = pl.pallas_call(
      _k4_body,
      out_shape=jax.ShapeDtypeStruct((N, D), jnp.float32),
  )(n_part, z2_part, p1_pad)
  return h_n


# speedup vs baseline: 35.3790x; 1.0102x over previous
import functools

import jax
import jax.numpy as jnp
from jax import lax
from jax.experimental import pallas as pl
from jax.experimental.pallas import tpu as pltpu
from jax.experimental.pallas import tpu_sc as plsc

N, M, E, D = 10000, 2000, 320000, 128
NC, NS, L = 2, 16, 16
NW = NC * NS
C = 128
SB = 8

TR1 = 10112
TR2 = 2048
ACC1 = 2048
ACC2 = 10112
OUT1 = ACC1
OUT2 = ACC2

K_CH = ((E + NW * C - 1) // (NW * C) + SB - 1) // SB * SB
EP = NW * C * K_CH
NSUP = K_CH // SB


def _sc_pass(tab_rows, sca_tab, scb_tab, gidx2d, sidx2d, *, acc_rows,
             out_rows, tr, nbuf, stage_tab, scb_is_ones=False, sb=SB):
  ur = scb_tab.shape[0]
  nsup = K_CH // sb
  opr = out_rows // NS
  spr = tr // NS
  nzb = acc_rows // C
  nzb_per = (nzb + NS - 1) // NS
  tab_sh_rows = tr if stage_tab else 16

  mesh = plsc.VectorSubcoreMesh(core_axis_name="c", subcore_axis_name="s",
                                num_cores=NC, num_subcores=NS)

  def body(tab_hbm, sca_hbm, scb_hbm, gidx_hbm, sidx_hbm,
           out_hbm, zout_hbm,
           idxg_v, idxs_v, rows_v, sca_v, scb_v, zred_v, acc_sh, tab_sh,
           isem, g0, g1, g2, g3, s0, s1, s2, s3):
    gsems = (g0, g1, g2, g3)[:nbuf]
    ssems = (s0, s1, s2, s3)[:nbuf]
    gtab = tab_sh if stage_tab else tab_hbm
    cid = lax.axis_index("c")
    sid = lax.axis_index("s")
    wid = cid * NS + sid

    if stage_tab:
      pltpu.sync_copy(tab_hbm.at[pl.ds(sid * spr, spr)],
                      tab_sh.at[pl.ds(sid * spr, spr)])
    pltpu.sync_copy(sca_hbm, sca_v)
    pltpu.sync_copy(scb_hbm, scb_v)

    zv = jnp.zeros((L,), jnp.float32)

    def zrow(r, _):
      for j in range(D // L):
        rows_v[0, r, pl.ds(j * L, L)] = zv
      return 0

    lax.fori_loop(0, C, zrow, 0)

    def zblk(i, _):
      b = i * NS + sid

      @pl.when(b < nzb)
      def _():
        pltpu.sync_copy(rows_v.at[0], acc_sh.at[pl.ds(b * C, C)])
      return 0

    lax.fori_loop(0, nzb_per, zblk, 0)

    plsc.subcore_barrier()

    crow0 = wid * K_CH

    pltpu.async_copy(gidx_hbm.at[pl.ds(crow0, sb)], idxg_v.at[0], isem)
    pltpu.async_copy(sidx_hbm.at[pl.ds(crow0, sb)], idxs_v.at[0], isem)

    def sup(u, zacc):
      pu = lax.rem(u, 2)
      cr = crow0 + u * sb
      pltpu.make_async_copy(gidx_hbm.at[pl.ds(cr, sb)],
                            idxg_v.at[pu], isem).wait()
      pltpu.make_async_copy(sidx_hbm.at[pl.ds(cr, sb)],
                            idxs_v.at[pu], isem).wait()
      dg = [None] * sb
      for j in range(sb + 1):
        if j < sb:
          b = j % nbuf
          if j >= nbuf:
            pltpu.make_async_copy(rows_v.at[b],
                                  acc_sh.at[idxs_v.at[pu, j - nbuf]],
                                  ssems[b]).wait()
          else:
            @pl.when(u > 0)
            def _():
              pltpu.make_async_copy(rows_v.at[b],
                                    acc_sh.at[idxs_v.at[pu, j]],
                                    ssems[b]).wait()
          dg[j] = pltpu.async_copy(gtab.at[idxg_v.at[pu, j]],
                                   rows_v.at[b], gsems[b])
        if j == nbuf:
          crn = crow0 + lax.rem(u + 1, nsup) * sb
          pltpu.async_copy(gidx_hbm.at[pl.ds(crn, sb)],
                           idxg_v.at[1 - pu], isem)
          pltpu.async_copy(sidx_hbm.at[pl.ds(crn, sb)],
                           idxs_v.at[1 - pu], isem)
        if j >= 1:
          jj = j - 1
          bb = jj % nbuf
          dg[jj].wait()
          pltpu.async_copy(rows_v.at[bb], acc_sh.at[idxs_v.at[pu, jj]],
                           ssems[bb], add=True)
          for h in range(C // L):
            ig = idxg_v[pu, jj, pl.ds(h * L, L)]
            va = plsc.load_gather(sca_v, [ig])
            if scb_is_ones:
              zacc = zacc + va
            else:
              isx = idxs_v[pu, jj, pl.ds(h * L, L)]
              vb = plsc.load_gather(scb_v, [isx])
              zacc = zacc + va * vb
      return zacc

    z = lax.fori_loop(0, nsup, sup, jnp.zeros((L,), jnp.float32))
    lp = (nsup - 1) % 2
    for j in range(sb - nbuf, sb):
      pltpu.make_async_copy(rows_v.at[j % nbuf],
                            acc_sh.at[idxs_v.at[lp, j]],
                            ssems[j % nbuf]).wait()
    pltpu.make_async_copy(gidx_hbm.at[pl.ds(crow0, sb)],
                          idxg_v.at[nsup % 2], isem).wait()
    pltpu.make_async_copy(sidx_hbm.at[pl.ds(crow0, sb)],
                          idxs_v.at[nsup % 2], isem).wait()

    plsc.subcore_barrier()

    zred_v[...] = z
    pltpu.sync_copy(zred_v, zout_hbm.at[wid])
    pltpu.sync_copy(acc_sh.at[pl.ds(sid * opr, opr)],
                    out_hbm.at[cid, pl.ds(sid * opr, opr)])

  call = pl.kernel(
      body,
      out_type=(jax.ShapeDtypeStruct((NC, out_rows, D), jnp.float32),
                jax.ShapeDtypeStruct((NW, L), jnp.float32)),
      mesh=mesh,
      compiler_params=pltpu.CompilerParams(needs_layout_passes=False),
      scratch_types=(
          pltpu.VMEM((2, sb, C), jnp.int32),
          pltpu.VMEM((2, sb, C), jnp.int32),
          pltpu.VMEM((nbuf, C, D), jnp.float32),
          pltpu.VMEM((tr,), jnp.float32),
          pltpu.VMEM((ur,), jnp.float32),
          pltpu.VMEM((L,), jnp.float32),
          pltpu.VMEM_SHARED((acc_rows, D), jnp.float32),
          pltpu.VMEM_SHARED((tab_sh_rows, D), jnp.float32),
          pltpu.SemaphoreType.DMA,
          pltpu.SemaphoreType.DMA,
          pltpu.SemaphoreType.DMA,
          pltpu.SemaphoreType.DMA,
          pltpu.SemaphoreType.DMA,
          pltpu.SemaphoreType.DMA,
          pltpu.SemaphoreType.DMA,
          pltpu.SemaphoreType.DMA,
          pltpu.SemaphoreType.DMA,
      ),
  )
  return call(tab_rows, sca_tab, scb_tab, gidx2d, sidx2d)


def _k12_body(x_ref, w_ref, b_ref, an_ref, y_ref, p1_ref):
  xt = jnp.dot(x_ref[...], w_ref[...],
               preferred_element_type=jnp.float32) + b_ref[...]
  a = jnp.dot(xt, an_ref[...], preferred_element_type=jnp.float32)
  p1 = jnp.exp(a - jnp.max(a))
  y_ref[: N, :] = p1 * xt
  y_ref[N:, :] = jnp.zeros((TR1 - N, D), jnp.float32)
  p1_ref[: N, :] = p1
  p1_ref[N:, :] = jnp.zeros((TR1 - N, 1), jnp.float32)


def _k3_body(hp_ref, z1_ref, ae_ref, z_ref, p2_ref):
  h = hp_ref[0, : M, :] + hp_ref[1, : M, :]
  z1 = jnp.sum(z1_ref[...])
  gu = jnp.dot(h, ae_ref[...], preferred_element_type=jnp.float32)
  p2 = jnp.exp((gu - jnp.max(gu)) / z1)
  z_ref[: M, :] = (p2 / z1) * h
  z_ref[M:, :] = jnp.zeros((TR2 - M, D), jnp.float32)
  p2_ref[: M, :] = p2
  p2_ref[M:, :] = jnp.zeros((TR2 - M, 1), jnp.float32)


def _k4_body(np_ref, z2_ref, p1_ref, o_ref):
  s = np_ref[0, : N, :] + np_ref[1, : N, :]
  z2 = jnp.sum(z2_ref[...])
  o_ref[...] = p1_ref[: N, :] * s * (1.0 / z2)


def kernel(x, in_node, in_hedge, W, b, attn_node, attn_edge):
  in_node = in_node.astype(jnp.int32)
  in_hedge = in_hedge.astype(jnp.int32)
  pad = EP - E
  ar = jnp.arange(pad, dtype=jnp.int32)
  g1 = jnp.concatenate([in_node, N + (ar % 16)]).reshape(EP // C, C)
  s1 = jnp.concatenate([in_hedge, ar % ACC1]).reshape(EP // C, C)
  g2 = jnp.concatenate([in_hedge, M + (ar % 16)]).reshape(EP // C, C)
  s2 = jnp.concatenate([in_node, ar % ACC2]).reshape(EP // C, C)

  y_pad, p1_pad = pl.pallas_call(
      _k12_body,
      out_shape=(jax.ShapeDtypeStruct((TR1, D), jnp.float32),
                 jax.ShapeDtypeStruct((TR1, 1), jnp.float32)),
  )(x, W, b.reshape(1, D), attn_node)

  ones_m = jnp.ones((TR2,), jnp.float32)
  h_part, z1_part = _sc_pass(y_pad, p1_pad.reshape(TR1), ones_m, g1, s1,
                             acc_rows=ACC1, out_rows=OUT1, tr=TR1,
                             nbuf=4, stage_tab=False, scb_is_ones=True,
                             sb=16)

  z_pad, p2_pad = pl.pallas_call(
      _k3_body,
      out_shape=(jax.ShapeDtypeStruct((TR2, D), jnp.float32),
                 jax.ShapeDtypeStruct((TR2, 1), jnp.float32)),
  )(h_part, z1_part, attn_edge)

  n_part, z2_part = _sc_pass(z_pad, p2_pad.reshape(TR2),
                             p1_pad.reshape(TR1), g2, s2,
                             acc_rows=ACC2, out_rows=OUT2, tr=TR2,
                             nbuf=2, stage_tab=False)

  h_n = pl.pallas_call(
      _k4_body,
      out_shape=jax.ShapeDtypeStruct((N, D), jnp.float32),
  )(n_part, z2_part, p1_pad)
  return h_n


# speedup vs baseline: 35.5615x; 1.0052x over previous
import functools

import jax
import jax.numpy as jnp
from jax import lax
from jax.experimental import pallas as pl
from jax.experimental.pallas import tpu as pltpu
from jax.experimental.pallas import tpu_sc as plsc

N, M, E, D = 10000, 2000, 320000, 128
NC, NS, L = 2, 16, 16
NW = NC * NS
C = 128
SB = 8

TR1 = 10112
TR2 = 2048
ACC1 = 2048
ACC2 = 10112
OUT1 = ACC1
OUT2 = ACC2

K_CH = ((E + NW * C - 1) // (NW * C) + SB - 1) // SB * SB
EP = NW * C * K_CH
NSUP = K_CH // SB


def _sc_pass(tab_rows, sca_tab, scb_tab, gidx2d, sidx2d, *, acc_rows,
             out_rows, tr, nbuf, stage_tab, scb_is_ones=False, sb=SB):
  ur = scb_tab.shape[0]
  nsup = K_CH // sb
  opr = out_rows // NS
  spr = tr // NS
  nzb = acc_rows // C
  nzb_per = (nzb + NS - 1) // NS
  tab_sh_rows = tr if stage_tab else 16

  mesh = plsc.VectorSubcoreMesh(core_axis_name="c", subcore_axis_name="s",
                                num_cores=NC, num_subcores=NS)

  def body(tab_hbm, sca_hbm, scb_hbm, gidx_hbm, sidx_hbm,
           out_hbm, zout_hbm,
           idxg_v, idxs_v, rows_v, sca_v, scb_v, zred_v, acc_sh, tab_sh,
           isem, g0, g1, g2, g3, g4, s0, s1, s2, s3, s4):
    gsems = (g0, g1, g2, g3, g4)[:nbuf]
    ssems = (s0, s1, s2, s3, s4)[:nbuf]
    gtab = tab_sh if stage_tab else tab_hbm
    cid = lax.axis_index("c")
    sid = lax.axis_index("s")
    wid = cid * NS + sid
    crow0 = wid * K_CH

    pltpu.async_copy(gidx_hbm.at[pl.ds(crow0, sb)], idxg_v.at[0], isem)
    pltpu.async_copy(sidx_hbm.at[pl.ds(crow0, sb)], idxs_v.at[0], isem)

    if stage_tab:
      pltpu.sync_copy(tab_hbm.at[pl.ds(sid * spr, spr)],
                      tab_sh.at[pl.ds(sid * spr, spr)])
    pltpu.sync_copy(sca_hbm, sca_v)
    pltpu.sync_copy(scb_hbm, scb_v)

    zv = jnp.zeros((L,), jnp.float32)

    def zrow(r, _):
      for j in range(D // L):
        rows_v[0, r, pl.ds(j * L, L)] = zv
      return 0

    lax.fori_loop(0, C, zrow, 0)

    def zblk(i, _):
      b = i * NS + sid

      @pl.when(b < nzb)
      def _():
        pltpu.sync_copy(rows_v.at[0], acc_sh.at[pl.ds(b * C, C)])
      return 0

    lax.fori_loop(0, nzb_per, zblk, 0)

    plsc.subcore_barrier()

    def sup(u, zacc):
      pu = lax.rem(u, 2)
      cr = crow0 + u * sb
      pltpu.make_async_copy(gidx_hbm.at[pl.ds(cr, sb)],
                            idxg_v.at[pu], isem).wait()
      pltpu.make_async_copy(sidx_hbm.at[pl.ds(cr, sb)],
                            idxs_v.at[pu], isem).wait()
      dg = [None] * sb
      for j in range(sb + 1):
        if j < sb:
          b = j % nbuf
          if j >= nbuf:
            pltpu.make_async_copy(rows_v.at[b],
                                  acc_sh.at[idxs_v.at[pu, j - nbuf]],
                                  ssems[b]).wait()
          else:
            @pl.when(u > 0)
            def _():
              pltpu.make_async_copy(rows_v.at[b],
                                    acc_sh.at[idxs_v.at[pu, j]],
                                    ssems[b]).wait()
          dg[j] = pltpu.async_copy(gtab.at[idxg_v.at[pu, j]],
                                   rows_v.at[b], gsems[b])
        if j == nbuf:
          crn = crow0 + lax.rem(u + 1, nsup) * sb
          pltpu.async_copy(gidx_hbm.at[pl.ds(crn, sb)],
                           idxg_v.at[1 - pu], isem)
          pltpu.async_copy(sidx_hbm.at[pl.ds(crn, sb)],
                           idxs_v.at[1 - pu], isem)
        if j >= 1:
          jj = j - 1
          bb = jj % nbuf
          dg[jj].wait()
          pltpu.async_copy(rows_v.at[bb], acc_sh.at[idxs_v.at[pu, jj]],
                           ssems[bb], add=True)
          for h in range(C // L):
            ig = idxg_v[pu, jj, pl.ds(h * L, L)]
            va = plsc.load_gather(sca_v, [ig])
            if scb_is_ones:
              zacc = zacc + va
            else:
              isx = idxs_v[pu, jj, pl.ds(h * L, L)]
              vb = plsc.load_gather(scb_v, [isx])
              zacc = zacc + va * vb
      return zacc

    z = lax.fori_loop(0, nsup, sup, jnp.zeros((L,), jnp.float32))
    lp = (nsup - 1) % 2
    for j in range(sb - nbuf, sb):
      pltpu.make_async_copy(rows_v.at[j % nbuf],
                            acc_sh.at[idxs_v.at[lp, j]],
                            ssems[j % nbuf]).wait()
    pltpu.make_async_copy(gidx_hbm.at[pl.ds(crow0, sb)],
                          idxg_v.at[nsup % 2], isem).wait()
    pltpu.make_async_copy(sidx_hbm.at[pl.ds(crow0, sb)],
                          idxs_v.at[nsup % 2], isem).wait()

    plsc.subcore_barrier()

    zred_v[...] = z
    pltpu.sync_copy(zred_v, zout_hbm.at[wid])
    pltpu.sync_copy(acc_sh.at[pl.ds(sid * opr, opr)],
                    out_hbm.at[cid, pl.ds(sid * opr, opr)])

  call = pl.kernel(
      body,
      out_type=(jax.ShapeDtypeStruct((NC, out_rows, D), jnp.float32),
                jax.ShapeDtypeStruct((NW, L), jnp.float32)),
      mesh=mesh,
      compiler_params=pltpu.CompilerParams(needs_layout_passes=False),
      scratch_types=(
          pltpu.VMEM((2, sb, C), jnp.int32),
          pltpu.VMEM((2, sb, C), jnp.int32),
          pltpu.VMEM((nbuf, C, D), jnp.float32),
          pltpu.VMEM((tr,), jnp.float32),
          pltpu.VMEM((ur,), jnp.float32),
          pltpu.VMEM((L,), jnp.float32),
          pltpu.VMEM_SHARED((acc_rows, D), jnp.float32),
          pltpu.VMEM_SHARED((tab_sh_rows, D), jnp.float32),
          pltpu.SemaphoreType.DMA,
          pltpu.SemaphoreType.DMA,
          pltpu.SemaphoreType.DMA,
          pltpu.SemaphoreType.DMA,
          pltpu.SemaphoreType.DMA,
          pltpu.SemaphoreType.DMA,
          pltpu.SemaphoreType.DMA,
          pltpu.SemaphoreType.DMA,
          pltpu.SemaphoreType.DMA,
          pltpu.SemaphoreType.DMA,
          pltpu.SemaphoreType.DMA,
      ),
  )
  return call(tab_rows, sca_tab, scb_tab, gidx2d, sidx2d)


def _k12_body(x_ref, w_ref, b_ref, an_ref, y_ref, p1_ref):
  xt = jnp.dot(x_ref[...], w_ref[...],
               preferred_element_type=jnp.float32) + b_ref[...]
  a = jnp.dot(xt, an_ref[...], preferred_element_type=jnp.float32)
  p1 = jnp.exp(a - jnp.max(a))
  y_ref[: N, :] = p1 * xt
  y_ref[N:, :] = jnp.zeros((TR1 - N, D), jnp.float32)
  p1_ref[: N, :] = p1
  p1_ref[N:, :] = jnp.zeros((TR1 - N, 1), jnp.float32)


def _k3_body(hp_ref, z1_ref, ae_ref, z_ref, p2_ref):
  h = hp_ref[0, : M, :] + hp_ref[1, : M, :]
  z1 = jnp.sum(z1_ref[...])
  gu = jnp.dot(h, ae_ref[...], preferred_element_type=jnp.float32)
  p2 = jnp.exp((gu - jnp.max(gu)) / z1)
  z_ref[: M, :] = (p2 / z1) * h
  z_ref[M:, :] = jnp.zeros((TR2 - M, D), jnp.float32)
  p2_ref[: M, :] = p2
  p2_ref[M:, :] = jnp.zeros((TR2 - M, 1), jnp.float32)


def _k4_body(np_ref, z2_ref, p1_ref, o_ref):
  s = np_ref[0, : N, :] + np_ref[1, : N, :]
  z2 = jnp.sum(z2_ref[...])
  o_ref[...] = p1_ref[: N, :] * s * (1.0 / z2)


def kernel(x, in_node, in_hedge, W, b, attn_node, attn_edge):
  in_node = in_node.astype(jnp.int32)
  in_hedge = in_hedge.astype(jnp.int32)
  pad = EP - E
  ar = jnp.arange(pad, dtype=jnp.int32)
  g1 = jnp.concatenate([in_node, N + (ar % 16)]).reshape(EP // C, C)
  s1 = jnp.concatenate([in_hedge, ar % ACC1]).reshape(EP // C, C)
  g2 = jnp.concatenate([in_hedge, M + (ar % 16)]).reshape(EP // C, C)
  s2 = jnp.concatenate([in_node, ar % ACC2]).reshape(EP // C, C)

  y_pad, p1_pad = pl.pallas_call(
      _k12_body,
      out_shape=(jax.ShapeDtypeStruct((TR1, D), jnp.float32),
                 jax.ShapeDtypeStruct((TR1, 1), jnp.float32)),
  )(x, W, b.reshape(1, D), attn_node)

  ones_m = jnp.ones((TR2,), jnp.float32)
  h_part, z1_part = _sc_pass(y_pad, p1_pad.reshape(TR1), ones_m, g1, s1,
                             acc_rows=ACC1, out_rows=OUT1, tr=TR1,
                             nbuf=5, stage_tab=False, scb_is_ones=True,
                             sb=16)

  z_pad, p2_pad = pl.pallas_call(
      _k3_body,
      out_shape=(jax.ShapeDtypeStruct((TR2, D), jnp.float32),
                 jax.ShapeDtypeStruct((TR2, 1), jnp.float32)),
  )(h_part, z1_part, attn_edge)

  n_part, z2_part = _sc_pass(z_pad, p2_pad.reshape(TR2),
                             p1_pad.reshape(TR1), g2, s2,
                             acc_rows=ACC2, out_rows=OUT2, tr=TR2,
                             nbuf=2, stage_tab=False)

  h_n = pl.pallas_call(
      _k4_body,
      out_shape=jax.ShapeDtypeStruct((N, D), jnp.float32),
  )(n_part, z2_part, p1_pad)
  return h_n
